# Initial kernel scaffold; baseline (speedup 1.0000x reference)
import jax
import jax.numpy as jnp
from jax.experimental import pallas as pl


def kernel(coordinates, hash_table):
    raise NotImplementedError("write your pallas kernel here")


# speedup vs baseline: 40.8619x; 40.8619x over previous
import dataclasses
import functools

import numpy as np
import jax
import jax.numpy as jnp
from jax import lax
from jax.experimental import pallas as pl
from jax.experimental.pallas import tpu as pltpu
from jax.experimental.pallas import tpu_sc as plsc

N_PTS = 524288
N_LEVELS = 16
TABLE = 1 << 19
MASK = TABLE - 1
BASE_RES = 16
FINEST_RES = 2048
_SCALE = float(np.exp((np.log(FINEST_RES) - np.log(BASE_RES)) / (N_LEVELS - 1)))
RES = [float(np.floor(BASE_RES * (_SCALE ** l))) for l in range(N_LEVELS)]
HA = 2654435761 - (1 << 32)
HB = 805459861

NW = 32
PTS_PER_TILE = N_PTS // NW
C = 256
CHUNKS = PTS_PER_TILE // C
G = C // 16
NSTREAM = (8 * C) // 128
TROWS = (N_LEVELS * TABLE * 2) // 16


def _compiler_params():
    cp = pltpu.CompilerParams()
    cp = dataclasses.replace(cp, needs_layout_passes=False,
                             use_tc_tiling_on_sc=False)
    return cp


@functools.partial(
    pl.kernel,
    out_type=jax.ShapeDtypeStruct((N_PTS, 2 * N_LEVELS), jnp.float32),
    mesh=plsc.VectorSubcoreMesh(core_axis_name="c", subcore_axis_name="s"),
    scratch_types=[
        pltpu.VMEM((C,), jnp.float32),
        pltpu.VMEM((C,), jnp.float32),
        pltpu.VMEM((C,), jnp.float32),
        pltpu.VMEM((2, 8, C), jnp.float32),
        pltpu.VMEM((2, 8, C), jnp.int32),
        pltpu.VMEM((2, NSTREAM, 128), jnp.int32),
        pltpu.VMEM((2, 8 * C, 16), jnp.float32),
        pltpu.VMEM((C, 2 * N_LEVELS), jnp.float32),
        pltpu.SemaphoreType.DMA,
        pltpu.SemaphoreType.DMA,
    ],
    compiler_params=_compiler_params(),
)
def _sc_encode(xh, yh, zh, th, oh, xv, yv, zv, wcb, subb, idxb, fb, ob,
               semA, semB):
    wid = lax.axis_index("c") * 16 + lax.axis_index("s")
    iota = lax.iota(jnp.int32, 16)
    col0 = jnp.zeros((16,), jnp.int32)
    sems = (semA, semB)
    hac = jnp.int32(HA)
    hbc = jnp.int32(HB)
    maskc = jnp.int32(MASK)

    def pass1(l, par):
        res = jnp.float32(RES[l])
        base_l = jnp.int32(l * TABLE)

        @pl.loop(0, G)
        def _(i):
            o = i * 16
            sl = pl.ds(o, 16)
            xs = xv[sl] * res
            ys = yv[sl] * res
            zs = zv[sl] * res
            xi = xs.astype(jnp.int32)
            yi = ys.astype(jnp.int32)
            zi = zs.astype(jnp.int32)
            wx1 = xs - xi.astype(jnp.float32)
            wy1 = ys - yi.astype(jnp.float32)
            wz1 = zs - zi.astype(jnp.float32)
            wz = (1.0 - wz1, wz1)
            hy0 = yi * hac
            hz0 = zi * hbc
            hx = (xi, xi + 1)
            hy = (hy0, hy0 + hac)
            hz = (hz0, hz0 + hbc)
            wxy = ((1.0 - wx1) * (1.0 - wy1), (1.0 - wx1) * wy1,
                   wx1 * (1.0 - wy1), wx1 * wy1)
            rr = i // 8
            cc = (i % 8) * 16
            for k in range(8):
                ox, oy, oz = (k >> 2) & 1, (k >> 1) & 1, k & 1
                e = ((hx[ox] ^ hy[oy] ^ hz[oz]) & maskc) + base_l
                idxb[par, 2 * k + rr, pl.ds(cc, 16)] = lax.shift_right_logical(e, 3)
                subb[par, k, sl] = e & 7
                wcb[par, k, sl] = wxy[2 * ox + oy] * wz[oz]

    def fire(par):
        sem = sems[par]

        @pl.loop(0, NSTREAM)
        def _(j):
            pltpu.make_async_copy(
                th.at[idxb.at[par, j]],
                fb.at[par, pl.ds(j * 128, 128)],
                sem,
            ).start()

    def drain(par):
        sem = sems[par]

        @pl.loop(0, NSTREAM)
        def _(j):
            pltpu.make_async_copy(
                th.at[idxb.at[par, j]],
                fb.at[par, pl.ds(j * 128, 128)],
                sem,
            ).wait()

    def pass2(l, par):
        f2 = fb.at[par]
        cl0 = col0 + (2 * l)
        cl1 = cl0 + 1

        @pl.loop(0, G)
        def _(i):
            o = i * 16
            sl = pl.ds(o, 16)
            rowb = iota + o
            acc0 = jnp.zeros((16,), jnp.float32)
            acc1 = jnp.zeros((16,), jnp.float32)
            for k in range(8):
                rk = rowb + (k * C)
                c0 = subb[par, k, sl] * 2
                f0 = plsc.load_gather(f2, [rk, c0])
                f1 = plsc.load_gather(f2, [rk, c0 + 1])
                wc = wcb[par, k, sl]
                acc0 = acc0 + f0 * wc
                acc1 = acc1 + f1 * wc
            acc0 = jnp.minimum(jnp.maximum(acc0, -10.0), 10.0)
            acc1 = jnp.minimum(jnp.maximum(acc1, -10.0), 10.0)
            plsc.store_scatter(ob, [rowb, cl0], acc0)
            plsc.store_scatter(ob, [rowb, cl1], acc1)

    tb = wid * PTS_PER_TILE

    @pl.loop(0, CHUNKS)
    def _(c):
        pbase = tb + c * C
        psl = pl.ds(pbase, C)
        pltpu.sync_copy(xh.at[psl], xv)
        pltpu.sync_copy(yh.at[psl], yv)
        pltpu.sync_copy(zh.at[psl], zv)
        pass1(0, 0)
        fire(0)
        for l in range(1, N_LEVELS):
            par = l & 1
            pass1(l, par)
            fire(par)
            drain(1 - par)
            pass2(l - 1, 1 - par)
        drain(1)
        pass2(N_LEVELS - 1, 1)
        pltpu.sync_copy(ob, oh.at[psl])


def kernel(coordinates, hash_table):
    x = coordinates[:, 0]
    y = coordinates[:, 1]
    z = coordinates[:, 2]
    table16 = hash_table.reshape(TROWS, 16)
    h = _sc_encode(x, y, z, table16)
    return (h, h, h)


# speedup vs baseline: 105.2833x; 2.5766x over previous
import dataclasses
import functools

import numpy as np
import jax
import jax.numpy as jnp
from jax import lax
from jax.experimental import pallas as pl
from jax.experimental.pallas import tpu as pltpu
from jax.experimental.pallas import tpu_sc as plsc

N_PTS = 524288
N_LEVELS = 16
TABLE = 1 << 19
MASK = TABLE - 1
BASE_RES = 16
FINEST_RES = 2048
_SCALE = float(np.exp((np.log(FINEST_RES) - np.log(BASE_RES)) / (N_LEVELS - 1)))
RES = [float(np.floor(BASE_RES * (_SCALE ** l))) for l in range(N_LEVELS)]
HA = 2654435761 - (1 << 32)
HB = 805459861

NW = 32
PTS_PER_TILE = N_PTS // NW
C = 128
CHUNKS = PTS_PER_TILE // C
G = C // 16
NSTREAM = (16 * C) // 128
TROWS = (N_LEVELS * TABLE * 2) // 16


def _compiler_params():
    cp = pltpu.CompilerParams()
    cp = dataclasses.replace(cp, needs_layout_passes=False,
                             use_tc_tiling_on_sc=False)
    return cp


@functools.partial(
    pl.kernel,
    out_type=jax.ShapeDtypeStruct((N_PTS, 2 * N_LEVELS), jnp.float32),
    mesh=plsc.VectorSubcoreMesh(core_axis_name="c", subcore_axis_name="s"),
    scratch_types=[
        pltpu.VMEM((C,), jnp.float32),
        pltpu.VMEM((C,), jnp.float32),
        pltpu.VMEM((C,), jnp.float32),
        pltpu.VMEM((2, 8, C), jnp.float32),
        pltpu.VMEM((2, 8, C), jnp.int32),
        pltpu.VMEM((2, NSTREAM, 128), jnp.int32),
        pltpu.VMEM((2, 16 * C, 16), jnp.float32),
        pltpu.VMEM((C, 2 * N_LEVELS), jnp.float32),
        pltpu.SemaphoreType.DMA,
        pltpu.SemaphoreType.DMA,
    ],
    compiler_params=_compiler_params(),
)
def _sc_encode(xh, yh, zh, th, oh, xv, yv, zv, wcb, subb, idxb, fb, ob,
               semA, semB):
    wid = lax.axis_index("c") * 16 + lax.axis_index("s")
    iota = lax.iota(jnp.int32, 16)
    col0 = jnp.zeros((16,), jnp.int32)
    sems = (semA, semB)
    hac = jnp.int32(HA)
    hbc = jnp.int32(HB)
    maskc = jnp.int32(MASK)

    def pass1(l, par):
        res = jnp.float32(RES[l])
        base_l = jnp.int32(l << 16)

        @pl.loop(0, G)
        def _(i):
            o = i * 16
            sl = pl.ds(o, 16)
            xs = xv[sl] * res
            ys = yv[sl] * res
            zs = zv[sl] * res
            xi = xs.astype(jnp.int32)
            yi = ys.astype(jnp.int32)
            zi = zs.astype(jnp.int32)
            wx1 = xs - xi.astype(jnp.float32)
            wy1 = ys - yi.astype(jnp.float32)
            wz1 = zs - zi.astype(jnp.float32)
            wz = (1.0 - wz1, wz1)
            hy0 = yi * hac
            hz0 = zi * hbc
            hx = (xi, xi + 1)
            hy = (hy0, hy0 + hac)
            hz = (hz0, hz0 + hbc)
            wxy = ((1.0 - wx1) * (1.0 - wy1), (1.0 - wx1) * wy1,
                   wx1 * (1.0 - wy1), wx1 * wy1)
            for k in range(8):
                ox, oy, oz = (k >> 2) & 1, (k >> 1) & 1, k & 1
                h = (hx[ox] ^ hy[oy] ^ hz[oz]) & maskc
                r0 = (base_l + lax.shift_left(lax.shift_right_logical(h, 7), 4)
                      + lax.shift_right_logical(h & 127, 4))
                idxb[par, k, sl] = r0
                idxb[par, 8 + k, sl] = r0 + 8
                subb[par, k, sl] = h & 15
                wcb[par, k, sl] = wxy[2 * ox + oy] * wz[oz]

    def fire(par):
        sem = sems[par]

        @pl.loop(0, NSTREAM)
        def _(j):
            pltpu.make_async_copy(
                th.at[idxb.at[par, j]],
                fb.at[par, pl.ds(j * 128, 128)],
                sem,
            ).start()

    def drain(par):
        sem = sems[par]

        @pl.loop(0, NSTREAM)
        def _(j):
            pltpu.make_async_copy(
                th.at[idxb.at[par, j]],
                fb.at[par, pl.ds(j * 128, 128)],
                sem,
            ).wait()

    def pass2(l, par):
        f2 = fb.at[par]
        cl0 = col0 + (2 * l)
        cl1 = cl0 + 1

        @pl.loop(0, G)
        def _(i):
            o = i * 16
            sl = pl.ds(o, 16)
            rowb = iota + o
            acc0 = jnp.zeros((16,), jnp.float32)
            acc1 = jnp.zeros((16,), jnp.float32)
            for k in range(8):
                rk0 = rowb + (k * C)
                sub = subb[par, k, sl]
                f0 = plsc.load_gather(f2, [rk0, sub])
                f1 = plsc.load_gather(f2, [rk0 + 8 * C, sub])
                wc = wcb[par, k, sl]
                acc0 = acc0 + f0 * wc
                acc1 = acc1 + f1 * wc
            acc0 = jnp.minimum(jnp.maximum(acc0, -10.0), 10.0)
            acc1 = jnp.minimum(jnp.maximum(acc1, -10.0), 10.0)
            plsc.store_scatter(ob, [rowb, cl0], acc0)
            plsc.store_scatter(ob, [rowb, cl1], acc1)

    tb = wid * PTS_PER_TILE

    @pl.loop(0, CHUNKS)
    def _(c):
        pbase = tb + c * C
        psl = pl.ds(pbase, C)
        pltpu.sync_copy(xh.at[psl], xv)
        pltpu.sync_copy(yh.at[psl], yv)
        pltpu.sync_copy(zh.at[psl], zv)
        pass1(0, 0)
        fire(0)
        for l in range(1, N_LEVELS):
            par = l & 1
            pass1(l, par)
            fire(par)
            drain(1 - par)
            pass2(l - 1, 1 - par)
        drain(1)
        pass2(N_LEVELS - 1, 1)
        pltpu.sync_copy(ob, oh.at[psl])


def kernel(coordinates, hash_table):
    x = coordinates[:, 0]
    y = coordinates[:, 1]
    z = coordinates[:, 2]
    tt = hash_table.reshape(N_LEVELS, 4096, 128, 2)
    tt = tt.transpose(0, 1, 3, 2)
    table16 = tt.reshape(TROWS, 16)
    h = _sc_encode(x, y, z, table16)
    return (h, h, h)
